# baseline (speedup 1.0000x reference)
import functools
import math

import jax
import jax.numpy as jnp
from jax import lax
from jax.experimental import pallas as pl
from jax.experimental.pallas import tpu as pltpu
from jax.experimental.pallas import tpu_sc as plsc

RBF_DIM = 16
RBF_DMAX = 15.0
NORM = 10.0


def _sigmoid(x):
    return 1.0 / (1.0 + jnp.exp(-x))


def _dot(a, b):
    return jnp.dot(a, b, preferred_element_type=jnp.float32)


def _edge_kernel(g1, g2, w0, w1, w2, whvm, whunit, pmat, wuvm, rmat,
                 b_out, wg, bg, out):
    S = 128
    V = 16
    g1v = g1[...]
    ps = g1v[:, S + 3 * V:S + 3 * V + 3]
    pd = g2[:, 0:3]
    xd = pd - ps
    d2 = jnp.sum(xd * xd, axis=1, keepdims=True)
    dist = jnp.sqrt(jnp.clip(d2, 1e-8))
    unit = xd / dist

    mu = (jnp.arange(RBF_DIM, dtype=jnp.int32).astype(jnp.float32)
          * (RBF_DMAX / (RBF_DIM - 1)))[None, :]
    sigma = RBF_DMAX / RBF_DIM
    rbf = jnp.exp(-(((dist - mu) / sigma) ** 2))

    coordf = g1v[:, S:S + 3 * V]
    vh = _dot(coordf, whvm[...]) + _dot(unit, whunit[...])
    sh = jnp.sqrt(jnp.clip(_dot(vh * vh, pmat[...]), 1e-8))

    lin = (_dot(g1v[:, :S], w0[...]) + _dot(rbf, w1[...])
           + _dot(sh, w2[...]) + b_out[...])
    feats = lin * _sigmoid(lin)
    gate = _sigmoid(_dot(feats, wg[...]) + bg[...])

    out[:, :S] = feats
    out[:, S:S + 3 * V] = _dot(gate, rmat[...]) * _dot(vh, wuvm[...])


def _node_kernel(agg0, agg1, sf, cf, w0, w1, whvm, pmat, wuvm, rmat, rtm,
                 b_out, wg, bg, g_msg, b_msg, g_upd, b_upd, out_s, out_v):
    S = 128
    V = 16
    agg = (agg0[...] + agg1[...]) * (1.0 / NORM)
    agg_s = agg[:, :S]
    mu = jnp.mean(agg_s, axis=1, keepdims=True)
    var = jnp.mean((agg_s - mu) ** 2, axis=1, keepdims=True)
    nf = (agg_s - mu) / jnp.sqrt(var + 1e-5) * g_msg[...] + b_msg[...]
    av = agg[:, S:S + 3 * V]
    rtv = rtm[...]
    nu = jnp.clip(_dot(av * av, rtv), 1e-8)
    vn = jnp.sqrt(jnp.mean(nu, axis=1, keepdims=True))

    s1 = sf[...] + nf
    v1 = cf[...] + av / vn

    vh = _dot(v1, whvm[...])
    sh = jnp.sqrt(jnp.clip(_dot(vh * vh, pmat[...]), 1e-8))
    lin = _dot(s1, w0[...]) + _dot(sh, w1[...]) + b_out[...]
    feats = lin * _sigmoid(lin)
    gate = _sigmoid(_dot(feats, wg[...]) + bg[...])
    uv = _dot(gate, rmat[...]) * _dot(vh, wuvm[...])

    s2 = s1 + feats
    v2 = v1 + uv
    mu2 = jnp.mean(s2, axis=1, keepdims=True)
    var2 = jnp.mean((s2 - mu2) ** 2, axis=1, keepdims=True)
    out_s[...] = (s2 - mu2) / jnp.sqrt(var2 + 1e-5) * g_upd[...] + b_upd[...]
    nu2 = jnp.clip(_dot(v2 * v2, rtv), 1e-8)
    vn2 = jnp.sqrt(jnp.mean(nu2, axis=1, keepdims=True))
    out_v[...] = v2 / vn2


def _sc_gather(tbl, posp, src, dst):
    N = tbl.shape[0]
    E = src.shape[0]
    D1 = tbl.shape[1]
    D2 = posp.shape[1]
    NW = 32
    EPW = E // NW
    CH = 128
    KFULL = EPW // CH
    TAIL = EPW - KFULL * CH

    mesh = plsc.VectorSubcoreMesh(core_axis_name="c", subcore_axis_name="s")

    @functools.partial(
        pl.kernel, mesh=mesh,
        compiler_params=pltpu.CompilerParams(use_tc_tiling_on_sc=False),
        out_type=[
            jax.ShapeDtypeStruct((E, D1), jnp.float32),
            jax.ShapeDtypeStruct((E, D2), jnp.float32),
        ],
        scratch_types=[
            pltpu.VMEM((EPW,), jnp.int32),
            pltpu.VMEM((EPW,), jnp.int32),
            pltpu.VMEM((2, CH, D1), jnp.float32),
            pltpu.VMEM((2, CH, D2), jnp.float32),
            pltpu.VMEM((TAIL, D1), jnp.float32),
            pltpu.VMEM((TAIL, D2), jnp.float32),
            pltpu.SemaphoreType.DMA,
            pltpu.SemaphoreType.DMA,
            pltpu.SemaphoreType.DMA,
            pltpu.SemaphoreType.DMA,
        ],
    )
    def gk(tbl_h, posp_h, src_h, dst_h, g1_h, g2_h,
           idxs_v, idxd_v, rows_v, prow_v, trow_v, tprow_v,
           sg0, sg1, sp0, sp1):
        wid = lax.axis_index("s") * 2 + lax.axis_index("c")
        base = pl.multiple_of(wid * EPW, 8)
        pltpu.sync_copy(src_h.at[pl.ds(base, EPW)], idxs_v)
        pltpu.sync_copy(dst_h.at[pl.ds(base, EPW)], idxd_v)
        sgs = [sg0, sg1]
        sps = [sp0, sp1]

        def body(i, carry):
            cps = []
            for b in range(2):
                k = i * 2 + b
                o = pl.multiple_of(k * CH, 8)
                cps.append(pltpu.async_copy(
                    tbl_h.at[idxs_v.at[pl.ds(o, CH)]], rows_v.at[b], sgs[b]))
                cps.append(pltpu.async_copy(
                    posp_h.at[idxd_v.at[pl.ds(o, CH)]], prow_v.at[b], sps[b]))
            for b in range(2):
                k = i * 2 + b
                oo = pl.multiple_of(base + k * CH, 8)
                cps[2 * b].wait()
                pltpu.sync_copy(rows_v.at[b], g1_h.at[pl.ds(oo, CH)])
                cps[2 * b + 1].wait()
                pltpu.sync_copy(prow_v.at[b], g2_h.at[pl.ds(oo, CH)])
            return carry

        lax.fori_loop(0, KFULL // 2, body, 0, unroll=False)

        ot = pl.multiple_of(KFULL * CH, 8)
        oot = pl.multiple_of(base + KFULL * CH, 8)
        pltpu.async_copy(tbl_h.at[idxs_v.at[pl.ds(ot, TAIL)]], trow_v, sg0).wait()
        pltpu.sync_copy(trow_v, g1_h.at[pl.ds(oot, TAIL)])
        pltpu.async_copy(posp_h.at[idxd_v.at[pl.ds(ot, TAIL)]], tprow_v, sp0).wait()
        pltpu.sync_copy(tprow_v, g2_h.at[pl.ds(oot, TAIL)])

    return gk(tbl, posp, src, dst)


def _sc_scatter(msg, dst, zinit):
    E, D = msg.shape
    NP = zinit.shape[0]
    NW = 32
    EPW = E // NW
    CH = 40
    KFULL = EPW // CH
    RPT = NP // 16

    mesh = plsc.VectorSubcoreMesh(core_axis_name="c", subcore_axis_name="s")

    @functools.partial(
        pl.kernel, mesh=mesh,
        compiler_params=pltpu.CompilerParams(use_tc_tiling_on_sc=False),
        out_type=jax.ShapeDtypeStruct((2, NP, D), jnp.float32),
        scratch_types=[
            pltpu.VMEM_SHARED((NP, D), jnp.float32),
            pltpu.VMEM((2, CH), jnp.int32),
            pltpu.VMEM((2, CH, D), jnp.float32),
            pltpu.SemaphoreType.DMA,
            pltpu.SemaphoreType.DMA,
        ],
    )
    def sk(msg_h, dst_h, zin_h, out_h,
           acc_s, idx_v, rows_v, s0, s1):
        cid = lax.axis_index("c")
        sid = lax.axis_index("s")
        wid = sid * 2 + cid
        base = pl.multiple_of(wid * EPW, 8)
        r0 = pl.multiple_of(sid * RPT, 8)

        pltpu.sync_copy(zin_h.at[pl.ds(r0, RPT)], acc_s.at[pl.ds(r0, RPT)])
        plsc.subcore_barrier()

        sems = [s0, s1]

        def body(i, carry):
            cps = []
            for b in range(2):
                k = i * 2 + b
                o = pl.multiple_of(base + k * CH, 8)
                cps.append(pltpu.async_copy(
                    dst_h.at[pl.ds(o, CH)], idx_v.at[b], sems[b]))
                cps.append(pltpu.async_copy(
                    msg_h.at[pl.ds(o, CH)], rows_v.at[b], sems[b]))
            for b in range(2):
                cps[2 * b].wait()
                cps[2 * b + 1].wait()
                pltpu.sync_copy(rows_v.at[b], acc_s.at[idx_v.at[b]], add=True)
            return carry

        lax.fori_loop(0, KFULL // 2, body, 0, unroll=False)

        plsc.subcore_barrier()
        pltpu.sync_copy(acc_s.at[pl.ds(r0, RPT)], out_h.at[cid, pl.ds(r0, RPT)])

    return sk(msg, dst, zinit)


def _block_weights(wh, wu, V, H):
    vi = jnp.arange(V)
    whvm = jnp.zeros((3 * V, 96), jnp.float32)
    wuvm = jnp.zeros((96, 3 * V), jnp.float32)
    pmat = jnp.zeros((96, H), jnp.float32)
    rmat = jnp.zeros((V, 3 * V), jnp.float32)
    whunit = jnp.zeros((3, 96), jnp.float32) if wh.shape[0] > V else None
    for c in range(3):
        whvm = whvm.at[3 * vi + c, 32 * c:32 * c + H].set(wh[:V])
        if whunit is not None:
            whunit = whunit.at[c, 32 * c:32 * c + H].set(wh[V])
        wuvm = wuvm.at[32 * c:32 * c + H, 3 * vi + c].set(wu)
        pmat = pmat.at[32 * c + jnp.arange(H), jnp.arange(H)].set(1.0)
        rmat = rmat.at[vi, 3 * vi + c].set(1.0)
    return whvm, whunit, pmat, wuvm, rmat, rmat.T


def kernel(scalar_feats, coord_feats, positions, edge_index, params):
    N, S = scalar_feats.shape
    V = coord_feats.shape[1]
    E = edge_index.shape[1]
    src = edge_index[0]
    dst = edge_index[1]

    coord_f = coord_feats.reshape(N, 3 * V)
    tbl = jnp.concatenate(
        [scalar_feats, coord_f, positions,
         jnp.zeros((N, 13), jnp.float32)], axis=1)
    posp = jnp.concatenate(
        [positions, jnp.zeros((N, 13), jnp.float32)], axis=1)

    g1, g2 = _sc_gather(tbl, posp, src, dst)

    pm = params['msg']
    w_out = pm['W_out']
    H1 = V + 1
    whvm_m, whunit_m, pmat_m, wuvm_m, rmat_m, _ = _block_weights(
        pm['Wh'], pm['Wu'], V, H1)
    TB = 2000
    msg = pl.pallas_call(
        _edge_kernel,
        grid=(E // TB,),
        in_specs=[
            pl.BlockSpec((TB, 192), lambda i: (i, 0)),
            pl.BlockSpec((TB, 16), lambda i: (i, 0)),
        ] + [pl.BlockSpec(s, lambda i: (0, 0)) for s in
             [(S, S), (RBF_DIM, S), (H1, S), (3 * V, 96), (3, 96),
              (96, H1), (96, 3 * V), (V, 3 * V), (1, S), (S, V), (1, V)]],
        out_specs=pl.BlockSpec((TB, S + 3 * V), lambda i: (i, 0)),
        out_shape=jax.ShapeDtypeStruct((E, S + 3 * V), jnp.float32),
    )(g1, g2,
      w_out[:S], w_out[S:S + RBF_DIM], w_out[S + RBF_DIM:],
      whvm_m, whunit_m, pmat_m, wuvm_m, rmat_m,
      pm['b_out'][None, :], pm['W_gate'], pm['b_gate'][None, :])

    NP = 16 * 632
    zinit = jnp.zeros((NP, S + 3 * V), jnp.float32)
    parts = _sc_scatter(msg, dst, zinit)
    agg0 = parts[0, :N]
    agg1 = parts[1, :N]

    pu = params['upd']
    wu_out = pu['W_out']
    whvm_u, _, pmat_u, wuvm_u, rmat_u, rtm_u = _block_weights(
        pu['Wh'], pu['Wu'], V, V)
    NB = 2000
    out_s, out_v = pl.pallas_call(
        _node_kernel,
        grid=(N // NB,),
        in_specs=[
            pl.BlockSpec((NB, S + 3 * V), lambda i: (i, 0)),
            pl.BlockSpec((NB, S + 3 * V), lambda i: (i, 0)),
            pl.BlockSpec((NB, S), lambda i: (i, 0)),
            pl.BlockSpec((NB, 3 * V), lambda i: (i, 0)),
        ] + [pl.BlockSpec(s, lambda i: (0, 0)) for s in
             [(S, S), (V, S), (3 * V, 96), (96, V), (96, 3 * V),
              (V, 3 * V), (3 * V, V), (1, S), (S, V), (1, V),
              (1, S), (1, S), (1, S), (1, S)]],
        out_specs=[
            pl.BlockSpec((NB, S), lambda i: (i, 0)),
            pl.BlockSpec((NB, 3 * V), lambda i: (i, 0)),
        ],
        out_shape=[
            jax.ShapeDtypeStruct((N, S), jnp.float32),
            jax.ShapeDtypeStruct((N, 3 * V), jnp.float32),
        ],
    )(agg0, agg1, scalar_feats, coord_f,
      wu_out[:S], wu_out[S:], whvm_u, pmat_u, wuvm_u, rmat_u, rtm_u,
      pu['b_out'][None, :], pu['W_gate'], pu['b_gate'][None, :],
      params['msg_ln']['gamma'][None, :], params['msg_ln']['beta'][None, :],
      params['upd_ln']['gamma'][None, :], params['upd_ln']['beta'][None, :])

    v2 = out_v.reshape(N, V, 3)
    return out_s, v2


# speedup vs baseline: 1.3107x; 1.3107x over previous
import functools
import math

import jax
import jax.numpy as jnp
from jax import lax
from jax.experimental import pallas as pl
from jax.experimental.pallas import tpu as pltpu
from jax.experimental.pallas import tpu_sc as plsc

RBF_DIM = 16
RBF_DMAX = 15.0
NORM = 10.0


def _sigmoid(x):
    return 1.0 / (1.0 + jnp.exp(-x))


def _dot(a, b):
    return jnp.dot(a, b, preferred_element_type=jnp.float32)


def _edge_kernel(g1, g2, w0, w1, w2, whvm, whunit, pmat, wuvm, rmat,
                 b_out, wg, bg, out):
    S = 128
    V = 16
    g1v = g1[...]
    ps = g1v[:, S + 3 * V:S + 3 * V + 3]
    pd = g2[:, 0:3]
    xd = pd - ps
    d2 = jnp.sum(xd * xd, axis=1, keepdims=True)
    dist = jnp.sqrt(jnp.clip(d2, 1e-8))
    unit = xd / dist

    mu = (jnp.arange(RBF_DIM, dtype=jnp.int32).astype(jnp.float32)
          * (RBF_DMAX / (RBF_DIM - 1)))[None, :]
    sigma = RBF_DMAX / RBF_DIM
    rbf = jnp.exp(-(((dist - mu) / sigma) ** 2))

    coordf = g1v[:, S:S + 3 * V]
    vh = _dot(coordf, whvm[...]) + _dot(unit, whunit[...])
    sh = jnp.sqrt(jnp.clip(_dot(vh * vh, pmat[...]), 1e-8))

    lin = (_dot(g1v[:, :S], w0[...]) + _dot(rbf, w1[...])
           + _dot(sh, w2[...]) + b_out[...])
    feats = lin * _sigmoid(lin)
    gate = _sigmoid(_dot(feats, wg[...]) + bg[...])

    out[:, :S] = feats
    out[:, S:S + 3 * V] = _dot(gate, rmat[...]) * _dot(vh, wuvm[...])


def _node_kernel(agg0, agg1, sf, cf, w0, w1, whvm, pmat, wuvm, rmat, rtm,
                 b_out, wg, bg, g_msg, b_msg, g_upd, b_upd, out_s, out_v):
    S = 128
    V = 16
    agg = (agg0[...] + agg1[...]) * (1.0 / NORM)
    agg_s = agg[:, :S]
    mu = jnp.mean(agg_s, axis=1, keepdims=True)
    var = jnp.mean((agg_s - mu) ** 2, axis=1, keepdims=True)
    nf = (agg_s - mu) / jnp.sqrt(var + 1e-5) * g_msg[...] + b_msg[...]
    av = agg[:, S:S + 3 * V]
    rtv = rtm[...]
    nu = jnp.clip(_dot(av * av, rtv), 1e-8)
    vn = jnp.sqrt(jnp.mean(nu, axis=1, keepdims=True))

    s1 = sf[...] + nf
    v1 = cf[...] + av / vn

    vh = _dot(v1, whvm[...])
    sh = jnp.sqrt(jnp.clip(_dot(vh * vh, pmat[...]), 1e-8))
    lin = _dot(s1, w0[...]) + _dot(sh, w1[...]) + b_out[...]
    feats = lin * _sigmoid(lin)
    gate = _sigmoid(_dot(feats, wg[...]) + bg[...])
    uv = _dot(gate, rmat[...]) * _dot(vh, wuvm[...])

    s2 = s1 + feats
    v2 = v1 + uv
    mu2 = jnp.mean(s2, axis=1, keepdims=True)
    var2 = jnp.mean((s2 - mu2) ** 2, axis=1, keepdims=True)
    out_s[...] = (s2 - mu2) / jnp.sqrt(var2 + 1e-5) * g_upd[...] + b_upd[...]
    nu2 = jnp.clip(_dot(v2 * v2, rtv), 1e-8)
    vn2 = jnp.sqrt(jnp.mean(nu2, axis=1, keepdims=True))
    out_v[...] = v2 / vn2


def _sc_gather(tbl, posp, src, dst):
    N = tbl.shape[0]
    E = src.shape[0]
    D1 = tbl.shape[1]
    D2 = posp.shape[1]
    NW = 32
    EPW = E // NW
    CH = 128
    KFULL = EPW // CH
    TAIL = EPW - KFULL * CH

    mesh = plsc.VectorSubcoreMesh(core_axis_name="c", subcore_axis_name="s")

    @functools.partial(
        pl.kernel, mesh=mesh,
        compiler_params=pltpu.CompilerParams(use_tc_tiling_on_sc=False),
        out_type=[
            jax.ShapeDtypeStruct((E, D1), jnp.float32),
            jax.ShapeDtypeStruct((E, D2), jnp.float32),
        ],
        scratch_types=[
            pltpu.VMEM((EPW,), jnp.int32),
            pltpu.VMEM((EPW,), jnp.int32),
            pltpu.VMEM((2, CH, D1), jnp.float32),
            pltpu.VMEM((2, CH, D2), jnp.float32),
            pltpu.VMEM((TAIL, D1), jnp.float32),
            pltpu.VMEM((TAIL, D2), jnp.float32),
            pltpu.SemaphoreType.DMA,
            pltpu.SemaphoreType.DMA,
            pltpu.SemaphoreType.DMA,
            pltpu.SemaphoreType.DMA,
        ],
    )
    def gk(tbl_h, posp_h, src_h, dst_h, g1_h, g2_h,
           idxs_v, idxd_v, rows_v, prow_v, trow_v, tprow_v,
           sg0, sg1, sp0, sp1):
        wid = lax.axis_index("s") * 2 + lax.axis_index("c")
        base = pl.multiple_of(wid * EPW, 8)
        pltpu.sync_copy(src_h.at[pl.ds(base, EPW)], idxs_v)
        pltpu.sync_copy(dst_h.at[pl.ds(base, EPW)], idxd_v)
        sgs = [sg0, sg1]
        sps = [sp0, sp1]

        def body(i, carry):
            cps = []
            for b in range(2):
                k = i * 2 + b
                o = pl.multiple_of(k * CH, 8)
                cps.append(pltpu.async_copy(
                    tbl_h.at[idxs_v.at[pl.ds(o, CH)]], rows_v.at[b], sgs[b]))
                cps.append(pltpu.async_copy(
                    posp_h.at[idxd_v.at[pl.ds(o, CH)]], prow_v.at[b], sps[b]))
            for b in range(2):
                k = i * 2 + b
                oo = pl.multiple_of(base + k * CH, 8)
                cps[2 * b].wait()
                pltpu.sync_copy(rows_v.at[b], g1_h.at[pl.ds(oo, CH)])
                cps[2 * b + 1].wait()
                pltpu.sync_copy(prow_v.at[b], g2_h.at[pl.ds(oo, CH)])
            return carry

        lax.fori_loop(0, KFULL // 2, body, 0, unroll=False)

        ot = pl.multiple_of(KFULL * CH, 8)
        oot = pl.multiple_of(base + KFULL * CH, 8)
        pltpu.async_copy(tbl_h.at[idxs_v.at[pl.ds(ot, TAIL)]], trow_v, sg0).wait()
        pltpu.sync_copy(trow_v, g1_h.at[pl.ds(oot, TAIL)])
        pltpu.async_copy(posp_h.at[idxd_v.at[pl.ds(ot, TAIL)]], tprow_v, sp0).wait()
        pltpu.sync_copy(tprow_v, g2_h.at[pl.ds(oot, TAIL)])

    return gk(tbl, posp, src, dst)


def _sc_scatter(msg, dst, zinit):
    E, D = msg.shape
    NP = zinit.shape[0]
    NW = 32
    EPW = E // NW
    CH = 40
    KFULL = EPW // CH
    RPT = NP // 16

    mesh = plsc.VectorSubcoreMesh(core_axis_name="c", subcore_axis_name="s")

    @functools.partial(
        pl.kernel, mesh=mesh,
        compiler_params=pltpu.CompilerParams(use_tc_tiling_on_sc=False),
        out_type=jax.ShapeDtypeStruct((2, NP, D), jnp.float32),
        scratch_types=[
            pltpu.VMEM_SHARED((NP, D), jnp.float32),
            pltpu.VMEM((2, CH), jnp.int32),
            pltpu.VMEM((2, CH, D), jnp.float32),
            pltpu.SemaphoreType.DMA,
            pltpu.SemaphoreType.DMA,
        ],
    )
    def sk(msg_h, dst_h, zin_h, out_h,
           acc_s, idx_v, rows_v, s0, s1):
        cid = lax.axis_index("c")
        sid = lax.axis_index("s")
        wid = sid * 2 + cid
        base = pl.multiple_of(wid * EPW, 8)
        r0 = pl.multiple_of(sid * RPT, 8)

        pltpu.sync_copy(zin_h.at[pl.ds(r0, RPT)], acc_s.at[pl.ds(r0, RPT)])
        plsc.subcore_barrier()

        sems = [s0, s1]

        def body(i, carry):
            cps = []
            for b in range(2):
                k = i * 2 + b
                o = pl.multiple_of(base + k * CH, 8)
                cps.append(pltpu.async_copy(
                    dst_h.at[pl.ds(o, CH)], idx_v.at[b], sems[b]))
                cps.append(pltpu.async_copy(
                    msg_h.at[pl.ds(o, CH)], rows_v.at[b], sems[b]))
            for b in range(2):
                cps[2 * b].wait()
                cps[2 * b + 1].wait()
                pltpu.sync_copy(rows_v.at[b], acc_s.at[idx_v.at[b]], add=True)
            return carry

        lax.fori_loop(0, KFULL // 2, body, 0, unroll=False)

        plsc.subcore_barrier()
        pltpu.sync_copy(acc_s.at[pl.ds(r0, RPT)], out_h.at[cid, pl.ds(r0, RPT)])

    return sk(msg, dst, zinit)


import numpy as _np


def _block_weights(wh, wu, V, H):
    L = _np.zeros((3, 3 * V, V), _np.float32)
    R = _np.zeros((3, H, 96), _np.float32)
    pmat = _np.zeros((96, H), _np.float32)
    rmat = _np.zeros((V, 3 * V), _np.float32)
    for c in range(3):
        L[c, 3 * _np.arange(V) + c, _np.arange(V)] = 1.0
        R[c, _np.arange(H), 32 * c + _np.arange(H)] = 1.0
        pmat[32 * c + _np.arange(H), _np.arange(H)] = 1.0
        rmat[_np.arange(V), 3 * _np.arange(V) + c] = 1.0
    whvm = jnp.einsum('cav,vh,chb->ab', L, wh[:V], R)
    whunit = (jnp.einsum('h,chb->cb', wh[V], R) if wh.shape[0] > V else None)
    wuvm = jnp.einsum('chb,hu,cau->ba', R, wu, jnp.asarray(L))
    return (whvm, whunit, jnp.asarray(pmat), wuvm,
            jnp.asarray(rmat), jnp.asarray(rmat.T))


def kernel(scalar_feats, coord_feats, positions, edge_index, params):
    N, S = scalar_feats.shape
    V = coord_feats.shape[1]
    E = edge_index.shape[1]
    src = edge_index[0]
    dst = edge_index[1]

    coord_f = coord_feats.reshape(N, 3 * V)
    tbl = jnp.concatenate(
        [scalar_feats, coord_f, positions,
         jnp.zeros((N, 13), jnp.float32)], axis=1)
    posp = jnp.concatenate(
        [positions, jnp.zeros((N, 13), jnp.float32)], axis=1)

    g1, g2 = _sc_gather(tbl, posp, src, dst)

    pm = params['msg']
    w_out = pm['W_out']
    H1 = V + 1
    whvm_m, whunit_m, pmat_m, wuvm_m, rmat_m, _ = _block_weights(
        pm['Wh'], pm['Wu'], V, H1)
    TB = 2000
    msg = pl.pallas_call(
        _edge_kernel,
        grid=(E // TB,),
        in_specs=[
            pl.BlockSpec((TB, 192), lambda i: (i, 0)),
            pl.BlockSpec((TB, 16), lambda i: (i, 0)),
        ] + [pl.BlockSpec(s, lambda i: (0, 0)) for s in
             [(S, S), (RBF_DIM, S), (H1, S), (3 * V, 96), (3, 96),
              (96, H1), (96, 3 * V), (V, 3 * V), (1, S), (S, V), (1, V)]],
        out_specs=pl.BlockSpec((TB, S + 3 * V), lambda i: (i, 0)),
        out_shape=jax.ShapeDtypeStruct((E, S + 3 * V), jnp.float32),
    )(g1, g2,
      w_out[:S], w_out[S:S + RBF_DIM], w_out[S + RBF_DIM:],
      whvm_m, whunit_m, pmat_m, wuvm_m, rmat_m,
      pm['b_out'][None, :], pm['W_gate'], pm['b_gate'][None, :])

    NP = 16 * 632
    zinit = jnp.zeros((NP, S + 3 * V), jnp.float32)
    parts = _sc_scatter(msg, dst, zinit)
    agg0 = parts[0, :N]
    agg1 = parts[1, :N]

    pu = params['upd']
    wu_out = pu['W_out']
    whvm_u, _, pmat_u, wuvm_u, rmat_u, rtm_u = _block_weights(
        pu['Wh'], pu['Wu'], V, V)
    NB = 2000
    out_s, out_v = pl.pallas_call(
        _node_kernel,
        grid=(N // NB,),
        in_specs=[
            pl.BlockSpec((NB, S + 3 * V), lambda i: (i, 0)),
            pl.BlockSpec((NB, S + 3 * V), lambda i: (i, 0)),
            pl.BlockSpec((NB, S), lambda i: (i, 0)),
            pl.BlockSpec((NB, 3 * V), lambda i: (i, 0)),
        ] + [pl.BlockSpec(s, lambda i: (0, 0)) for s in
             [(S, S), (V, S), (3 * V, 96), (96, V), (96, 3 * V),
              (V, 3 * V), (3 * V, V), (1, S), (S, V), (1, V),
              (1, S), (1, S), (1, S), (1, S)]],
        out_specs=[
            pl.BlockSpec((NB, S), lambda i: (i, 0)),
            pl.BlockSpec((NB, 3 * V), lambda i: (i, 0)),
        ],
        out_shape=[
            jax.ShapeDtypeStruct((N, S), jnp.float32),
            jax.ShapeDtypeStruct((N, 3 * V), jnp.float32),
        ],
    )(agg0, agg1, scalar_feats, coord_f,
      wu_out[:S], wu_out[S:], whvm_u, pmat_u, wuvm_u, rmat_u, rtm_u,
      pu['b_out'][None, :], pu['W_gate'], pu['b_gate'][None, :],
      params['msg_ln']['gamma'][None, :], params['msg_ln']['beta'][None, :],
      params['upd_ln']['gamma'][None, :], params['upd_ln']['beta'][None, :])

    v2 = out_v.reshape(N, V, 3)
    return out_s, v2


# speedup vs baseline: 1.6137x; 1.2312x over previous
import functools
import math

import jax
import jax.numpy as jnp
from jax import lax
from jax.experimental import pallas as pl
from jax.experimental.pallas import tpu as pltpu
from jax.experimental.pallas import tpu_sc as plsc

RBF_DIM = 16
RBF_DMAX = 15.0
NORM = 10.0


def _sigmoid(x):
    return 1.0 / (1.0 + jnp.exp(-x))


def _dot(a, b):
    return jnp.dot(a, b, preferred_element_type=jnp.float32)


def _edge_kernel(g1, g2, ones3, w0, w1, w2, whvm, whunit, pmat, wuvm, rmat,
                 b_out, wg, bg, out):
    S = 128
    V = 16
    g1v = g1[...]
    ps = g1v[:, S + 3 * V:S + 3 * V + 3]
    pd = g2[:, 0:3]
    xd = pd - ps
    d2r = jnp.clip(_dot(xd * xd, ones3[...]), 1e-8)
    invr = jax.lax.rsqrt(d2r)
    unit = xd * invr[:, 0:3]

    mu = (jnp.arange(RBF_DIM, dtype=jnp.int32).astype(jnp.float32)
          * (RBF_DMAX / (RBF_DIM - 1)))[None, :]
    sigma = RBF_DMAX / RBF_DIM
    rbf = jnp.exp(-(((d2r * invr - mu) / sigma) ** 2))

    coordf = g1v[:, S:S + 3 * V]
    vh = _dot(coordf, whvm[...]) + _dot(unit, whunit[...])
    sh = jnp.sqrt(jnp.clip(_dot(vh * vh, pmat[...]), 1e-8))

    lin = (_dot(g1v[:, :S], w0[...]) + _dot(rbf, w1[...])
           + _dot(sh, w2[...]) + b_out[...])
    feats = lin * _sigmoid(lin)
    gate = _sigmoid(_dot(feats, wg[...]) + bg[...])

    out[:, :S] = feats
    out[:, S:S + 3 * V] = _dot(gate, rmat[...]) * _dot(vh, wuvm[...])


def _node_kernel(agg0, agg1, sf, cf, w0, w1, whvm, pmat, wuvm, rmat, rtm,
                 b_out, wg, bg, g_msg, b_msg, g_upd, b_upd, out_s, out_v):
    S = 128
    V = 16
    agg = (agg0[...] + agg1[...]) * (1.0 / NORM)
    agg_s = agg[:, :S]
    mu = jnp.mean(agg_s, axis=1, keepdims=True)
    var = jnp.mean((agg_s - mu) ** 2, axis=1, keepdims=True)
    nf = (agg_s - mu) / jnp.sqrt(var + 1e-5) * g_msg[...] + b_msg[...]
    av = agg[:, S:S + 3 * V]
    rtv = rtm[...]
    nu = jnp.clip(_dot(av * av, rtv), 1e-8)
    vn = jnp.sqrt(jnp.mean(nu, axis=1, keepdims=True))

    s1 = sf[...] + nf
    v1 = cf[...] + av / vn

    vh = _dot(v1, whvm[...])
    sh = jnp.sqrt(jnp.clip(_dot(vh * vh, pmat[...]), 1e-8))
    lin = _dot(s1, w0[...]) + _dot(sh, w1[...]) + b_out[...]
    feats = lin * _sigmoid(lin)
    gate = _sigmoid(_dot(feats, wg[...]) + bg[...])
    uv = _dot(gate, rmat[...]) * _dot(vh, wuvm[...])

    s2 = s1 + feats
    v2 = v1 + uv
    mu2 = jnp.mean(s2, axis=1, keepdims=True)
    var2 = jnp.mean((s2 - mu2) ** 2, axis=1, keepdims=True)
    out_s[...] = (s2 - mu2) / jnp.sqrt(var2 + 1e-5) * g_upd[...] + b_upd[...]
    nu2 = jnp.clip(_dot(v2 * v2, rtv), 1e-8)
    vn2 = jnp.sqrt(jnp.mean(nu2, axis=1, keepdims=True))
    out_v[...] = v2 / vn2


def _sc_gather(tbl, posp, src, dst):
    N = tbl.shape[0]
    E = src.shape[0]
    D1 = tbl.shape[1]
    D2 = posp.shape[1]
    NW = 32
    EPW = E // NW
    CH = 128
    KFULL = EPW // CH
    TAIL = EPW - KFULL * CH

    mesh = plsc.VectorSubcoreMesh(core_axis_name="c", subcore_axis_name="s")

    @functools.partial(
        pl.kernel, mesh=mesh,
        compiler_params=pltpu.CompilerParams(use_tc_tiling_on_sc=False),
        out_type=[
            jax.ShapeDtypeStruct((E, D1), jnp.float32),
            jax.ShapeDtypeStruct((E, D2), jnp.float32),
        ],
        scratch_types=[
            pltpu.VMEM((EPW,), jnp.int32),
            pltpu.VMEM((EPW,), jnp.int32),
            pltpu.VMEM((2, CH, D1), jnp.float32),
            pltpu.VMEM((2, CH, D2), jnp.float32),
            pltpu.VMEM((TAIL, D1), jnp.float32),
            pltpu.VMEM((TAIL, D2), jnp.float32),
            pltpu.SemaphoreType.DMA,
            pltpu.SemaphoreType.DMA,
            pltpu.SemaphoreType.DMA,
            pltpu.SemaphoreType.DMA,
        ],
    )
    def gk(tbl_h, posp_h, src_h, dst_h, g1_h, g2_h,
           idxs_v, idxd_v, rows_v, prow_v, trow_v, tprow_v,
           sg0, sg1, sp0, sp1):
        wid = lax.axis_index("s") * 2 + lax.axis_index("c")
        base = pl.multiple_of(wid * EPW, 8)
        pltpu.sync_copy(src_h.at[pl.ds(base, EPW)], idxs_v)
        pltpu.sync_copy(dst_h.at[pl.ds(base, EPW)], idxd_v)
        sgs = [sg0, sg1]
        sps = [sp0, sp1]

        def body(i, carry):
            cps = []
            for b in range(2):
                k = i * 2 + b
                o = pl.multiple_of(k * CH, 8)
                cps.append(pltpu.async_copy(
                    tbl_h.at[idxs_v.at[pl.ds(o, CH)]], rows_v.at[b], sgs[b]))
                cps.append(pltpu.async_copy(
                    posp_h.at[idxd_v.at[pl.ds(o, CH)]], prow_v.at[b], sps[b]))
            for b in range(2):
                k = i * 2 + b
                oo = pl.multiple_of(base + k * CH, 8)
                cps[2 * b].wait()
                pltpu.sync_copy(rows_v.at[b], g1_h.at[pl.ds(oo, CH)])
                cps[2 * b + 1].wait()
                pltpu.sync_copy(prow_v.at[b], g2_h.at[pl.ds(oo, CH)])
            return carry

        lax.fori_loop(0, KFULL // 2, body, 0, unroll=False)

        ot = pl.multiple_of(KFULL * CH, 8)
        oot = pl.multiple_of(base + KFULL * CH, 8)
        pltpu.async_copy(tbl_h.at[idxs_v.at[pl.ds(ot, TAIL)]], trow_v, sg0).wait()
        pltpu.sync_copy(trow_v, g1_h.at[pl.ds(oot, TAIL)])
        pltpu.async_copy(posp_h.at[idxd_v.at[pl.ds(ot, TAIL)]], tprow_v, sp0).wait()
        pltpu.sync_copy(tprow_v, g2_h.at[pl.ds(oot, TAIL)])

    return gk(tbl, posp, src, dst)


def _sc_scatter(msg, dst, zinit):
    E, D = msg.shape
    NP = zinit.shape[0]
    NW = 32
    EPW = E // NW
    CH = 40
    KFULL = EPW // CH
    RPT = NP // 16

    mesh = plsc.VectorSubcoreMesh(core_axis_name="c", subcore_axis_name="s")

    @functools.partial(
        pl.kernel, mesh=mesh,
        compiler_params=pltpu.CompilerParams(use_tc_tiling_on_sc=False),
        out_type=jax.ShapeDtypeStruct((2, NP, D), jnp.float32),
        scratch_types=[
            pltpu.VMEM_SHARED((NP, D), jnp.float32),
            pltpu.VMEM((2, CH), jnp.int32),
            pltpu.VMEM((2, CH, D), jnp.float32),
            pltpu.SemaphoreType.DMA,
            pltpu.SemaphoreType.DMA,
        ],
    )
    def sk(msg_h, dst_h, zin_h, out_h,
           acc_s, idx_v, rows_v, s0, s1):
        cid = lax.axis_index("c")
        sid = lax.axis_index("s")
        wid = sid * 2 + cid
        base = pl.multiple_of(wid * EPW, 8)
        r0 = pl.multiple_of(sid * RPT, 8)

        pltpu.sync_copy(zin_h.at[pl.ds(r0, RPT)], acc_s.at[pl.ds(r0, RPT)])
        plsc.subcore_barrier()

        sems = [s0, s1]

        def body(i, carry):
            cps = []
            for b in range(2):
                k = i * 2 + b
                o = pl.multiple_of(base + k * CH, 8)
                cps.append(pltpu.async_copy(
                    dst_h.at[pl.ds(o, CH)], idx_v.at[b], sems[b]))
                cps.append(pltpu.async_copy(
                    msg_h.at[pl.ds(o, CH)], rows_v.at[b], sems[b]))
            for b in range(2):
                cps[2 * b].wait()
                cps[2 * b + 1].wait()
                pltpu.sync_copy(rows_v.at[b], acc_s.at[idx_v.at[b]], add=True)
            return carry

        lax.fori_loop(0, KFULL // 2, body, 0, unroll=False)

        plsc.subcore_barrier()
        pltpu.sync_copy(acc_s.at[pl.ds(r0, RPT)], out_h.at[cid, pl.ds(r0, RPT)])

    return sk(msg, dst, zinit)


import numpy as _np


def _block_weights(wh, wu, V, H):
    L = _np.zeros((3, 3 * V, V), _np.float32)
    R = _np.zeros((3, H, 96), _np.float32)
    pmat = _np.zeros((96, H), _np.float32)
    rmat = _np.zeros((V, 3 * V), _np.float32)
    for c in range(3):
        L[c, 3 * _np.arange(V) + c, _np.arange(V)] = 1.0
        R[c, _np.arange(H), 32 * c + _np.arange(H)] = 1.0
        pmat[32 * c + _np.arange(H), _np.arange(H)] = 1.0
        rmat[_np.arange(V), 3 * _np.arange(V) + c] = 1.0
    whvm = jnp.einsum('cav,vh,chb->ab', L, wh[:V], R)
    whunit = (jnp.einsum('h,chb->cb', wh[V], R) if wh.shape[0] > V else None)
    wuvm = jnp.einsum('chb,hu,cau->ba', R, wu, jnp.asarray(L))
    return (whvm, whunit, jnp.asarray(pmat), wuvm,
            jnp.asarray(rmat), jnp.asarray(rmat.T))


def kernel(scalar_feats, coord_feats, positions, edge_index, params):
    N, S = scalar_feats.shape
    V = coord_feats.shape[1]
    E = edge_index.shape[1]
    src = edge_index[0]
    dst = edge_index[1]

    coord_f = coord_feats.reshape(N, 3 * V)
    tbl = jnp.concatenate(
        [scalar_feats, coord_f, positions,
         jnp.zeros((N, 13), jnp.float32)], axis=1)
    posp = jnp.concatenate(
        [positions, jnp.zeros((N, 13), jnp.float32)], axis=1)

    g1, g2 = _sc_gather(tbl, posp, src, dst)

    pm = params['msg']
    w_out = pm['W_out']
    H1 = V + 1
    whvm_m, whunit_m, pmat_m, wuvm_m, rmat_m, _ = _block_weights(
        pm['Wh'], pm['Wu'], V, H1)
    TB = 4000
    msg = pl.pallas_call(
        _edge_kernel,
        grid=(E // TB,),
        in_specs=[
            pl.BlockSpec((TB, 192), lambda i: (i, 0)),
            pl.BlockSpec((TB, 16), lambda i: (i, 0)),
        ] + [pl.BlockSpec(s, lambda i: (0, 0)) for s in
             [(3, V), (S, S), (RBF_DIM, S), (H1, S), (3 * V, 96), (3, 96),
              (96, H1), (96, 3 * V), (V, 3 * V), (1, S), (S, V), (1, V)]],
        out_specs=pl.BlockSpec((TB, S + 3 * V), lambda i: (i, 0)),
        out_shape=jax.ShapeDtypeStruct((E, S + 3 * V), jnp.float32),
    )(g1, g2, jnp.asarray(_np.ones((3, V), _np.float32)),
      w_out[:S], w_out[S:S + RBF_DIM], w_out[S + RBF_DIM:],
      whvm_m, whunit_m, pmat_m, wuvm_m, rmat_m,
      pm['b_out'][None, :], pm['W_gate'], pm['b_gate'][None, :])

    NP = 16 * 632
    zinit = jnp.zeros((NP, S + 3 * V), jnp.float32)
    parts = _sc_scatter(msg, dst, zinit)
    agg0 = parts[0, :N]
    agg1 = parts[1, :N]

    pu = params['upd']
    wu_out = pu['W_out']
    whvm_u, _, pmat_u, wuvm_u, rmat_u, rtm_u = _block_weights(
        pu['Wh'], pu['Wu'], V, V)
    NB = 2000
    out_s, out_v = pl.pallas_call(
        _node_kernel,
        grid=(N // NB,),
        in_specs=[
            pl.BlockSpec((NB, S + 3 * V), lambda i: (i, 0)),
            pl.BlockSpec((NB, S + 3 * V), lambda i: (i, 0)),
            pl.BlockSpec((NB, S), lambda i: (i, 0)),
            pl.BlockSpec((NB, 3 * V), lambda i: (i, 0)),
        ] + [pl.BlockSpec(s, lambda i: (0, 0)) for s in
             [(S, S), (V, S), (3 * V, 96), (96, V), (96, 3 * V),
              (V, 3 * V), (3 * V, V), (1, S), (S, V), (1, V),
              (1, S), (1, S), (1, S), (1, S)]],
        out_specs=[
            pl.BlockSpec((NB, S), lambda i: (i, 0)),
            pl.BlockSpec((NB, 3 * V), lambda i: (i, 0)),
        ],
        out_shape=[
            jax.ShapeDtypeStruct((N, S), jnp.float32),
            jax.ShapeDtypeStruct((N, 3 * V), jnp.float32),
        ],
    )(agg0, agg1, scalar_feats, coord_f,
      wu_out[:S], wu_out[S:], whvm_u, pmat_u, wuvm_u, rmat_u, rtm_u,
      pu['b_out'][None, :], pu['W_gate'], pu['b_gate'][None, :],
      params['msg_ln']['gamma'][None, :], params['msg_ln']['beta'][None, :],
      params['upd_ln']['gamma'][None, :], params['upd_ln']['beta'][None, :])

    v2 = out_v.reshape(N, V, 3)
    return out_s, v2


# speedup vs baseline: 1.6375x; 1.0147x over previous
import functools
import math

import jax
import jax.numpy as jnp
from jax import lax
from jax.experimental import pallas as pl
from jax.experimental.pallas import tpu as pltpu
from jax.experimental.pallas import tpu_sc as plsc

RBF_DIM = 16
RBF_DMAX = 15.0
NORM = 10.0


def _sigmoid(x):
    return 1.0 / (1.0 + jnp.exp(-x))


def _dot(a, b):
    return jnp.dot(a, b, preferred_element_type=jnp.float32)


def _edge_kernel(g1, g2, ones3, w0, w1, w2, whvm, whunit, pmat, wuvm, rmat,
                 b_out, wg, bg, out):
    S = 128
    V = 16
    g1v = g1[...]
    ps = g1v[:, S + 3 * V:S + 3 * V + 3]
    pd = g2[:, 0:3]
    xd = pd - ps
    d2r = jnp.clip(_dot(xd * xd, ones3[...]), 1e-8)
    invr = jax.lax.rsqrt(d2r)
    unit = xd * invr[:, 0:3]

    mu = (jnp.arange(RBF_DIM, dtype=jnp.int32).astype(jnp.float32)
          * (RBF_DMAX / (RBF_DIM - 1)))[None, :]
    sigma = RBF_DMAX / RBF_DIM
    rbf = jnp.exp(-(((d2r * invr - mu) / sigma) ** 2))

    coordf = g1v[:, S:S + 3 * V]
    vh = _dot(coordf, whvm[...]) + _dot(unit, whunit[...])
    sh = jnp.sqrt(jnp.clip(_dot(vh * vh, pmat[...]), 1e-8))

    lin = (_dot(g1v[:, :S], w0[...]) + _dot(rbf, w1[...])
           + _dot(sh, w2[...]) + b_out[...])
    feats = lin * _sigmoid(lin)
    gate = _sigmoid(_dot(feats, wg[...]) + bg[...])

    out[:, :S] = feats
    out[:, S:S + 3 * V] = _dot(gate, rmat[...]) * _dot(vh, wuvm[...])


def _node_kernel(agg0, agg1, sf, cf, w0, w1, whvm, pmat, wuvm, rmat, rtm,
                 b_out, wg, bg, g_msg, b_msg, g_upd, b_upd, out_s, out_v):
    S = 128
    V = 16
    agg = (agg0[...][0] + agg1[...][0]) * (1.0 / NORM)
    agg_s = agg[:, :S]
    mu = jnp.mean(agg_s, axis=1, keepdims=True)
    var = jnp.mean((agg_s - mu) ** 2, axis=1, keepdims=True)
    nf = (agg_s - mu) / jnp.sqrt(var + 1e-5) * g_msg[...] + b_msg[...]
    av = agg[:, S:S + 3 * V]
    rtv = rtm[...]
    nu = jnp.clip(_dot(av * av, rtv), 1e-8)
    vn = jnp.sqrt(jnp.mean(nu, axis=1, keepdims=True))

    s1 = sf[...] + nf
    v1 = cf[...] + av / vn

    vh = _dot(v1, whvm[...])
    sh = jnp.sqrt(jnp.clip(_dot(vh * vh, pmat[...]), 1e-8))
    lin = _dot(s1, w0[...]) + _dot(sh, w1[...]) + b_out[...]
    feats = lin * _sigmoid(lin)
    gate = _sigmoid(_dot(feats, wg[...]) + bg[...])
    uv = _dot(gate, rmat[...]) * _dot(vh, wuvm[...])

    s2 = s1 + feats
    v2 = v1 + uv
    mu2 = jnp.mean(s2, axis=1, keepdims=True)
    var2 = jnp.mean((s2 - mu2) ** 2, axis=1, keepdims=True)
    out_s[...] = (s2 - mu2) / jnp.sqrt(var2 + 1e-5) * g_upd[...] + b_upd[...]
    nu2 = jnp.clip(_dot(v2 * v2, rtv), 1e-8)
    vn2 = jnp.sqrt(jnp.mean(nu2, axis=1, keepdims=True))
    out_v[...] = v2 / vn2


def _sc_gather(tbl, posp, src, dst):
    N = tbl.shape[0]
    E = src.shape[0]
    D1 = tbl.shape[1]
    D2 = posp.shape[1]
    NW = 32
    EPW = E // NW
    CH = 128
    KFULL = EPW // CH
    TAIL = EPW - KFULL * CH

    mesh = plsc.VectorSubcoreMesh(core_axis_name="c", subcore_axis_name="s")

    @functools.partial(
        pl.kernel, mesh=mesh,
        compiler_params=pltpu.CompilerParams(use_tc_tiling_on_sc=False),
        out_type=[
            jax.ShapeDtypeStruct((E, D1), jnp.float32),
            jax.ShapeDtypeStruct((E, D2), jnp.float32),
        ],
        scratch_types=[
            pltpu.VMEM((EPW,), jnp.int32),
            pltpu.VMEM((EPW,), jnp.int32),
            pltpu.VMEM((2, CH, D1), jnp.float32),
            pltpu.VMEM((2, CH, D2), jnp.float32),
            pltpu.VMEM((TAIL, D1), jnp.float32),
            pltpu.VMEM((TAIL, D2), jnp.float32),
            pltpu.SemaphoreType.DMA,
            pltpu.SemaphoreType.DMA,
            pltpu.SemaphoreType.DMA,
            pltpu.SemaphoreType.DMA,
        ],
    )
    def gk(tbl_h, posp_h, src_h, dst_h, g1_h, g2_h,
           idxs_v, idxd_v, rows_v, prow_v, trow_v, tprow_v,
           sg0, sg1, sp0, sp1):
        wid = lax.axis_index("s") * 2 + lax.axis_index("c")
        base = pl.multiple_of(wid * EPW, 8)
        pltpu.sync_copy(src_h.at[pl.ds(base, EPW)], idxs_v)
        pltpu.sync_copy(dst_h.at[pl.ds(base, EPW)], idxd_v)
        sgs = [sg0, sg1]
        sps = [sp0, sp1]

        def body(i, carry):
            cps = []
            for b in range(2):
                k = i * 2 + b
                o = pl.multiple_of(k * CH, 8)
                cps.append(pltpu.async_copy(
                    tbl_h.at[idxs_v.at[pl.ds(o, CH)]], rows_v.at[b], sgs[b]))
                cps.append(pltpu.async_copy(
                    posp_h.at[idxd_v.at[pl.ds(o, CH)]], prow_v.at[b], sps[b]))
            for b in range(2):
                k = i * 2 + b
                oo = pl.multiple_of(base + k * CH, 8)
                cps[2 * b].wait()
                pltpu.sync_copy(rows_v.at[b], g1_h.at[pl.ds(oo, CH)])
                cps[2 * b + 1].wait()
                pltpu.sync_copy(prow_v.at[b], g2_h.at[pl.ds(oo, CH)])
            return carry

        lax.fori_loop(0, KFULL // 2, body, 0, unroll=False)

        ot = pl.multiple_of(KFULL * CH, 8)
        oot = pl.multiple_of(base + KFULL * CH, 8)
        pltpu.async_copy(tbl_h.at[idxs_v.at[pl.ds(ot, TAIL)]], trow_v, sg0).wait()
        pltpu.sync_copy(trow_v, g1_h.at[pl.ds(oot, TAIL)])
        pltpu.async_copy(posp_h.at[idxd_v.at[pl.ds(ot, TAIL)]], tprow_v, sp0).wait()
        pltpu.sync_copy(tprow_v, g2_h.at[pl.ds(oot, TAIL)])

    return gk(tbl, posp, src, dst)


def _sc_scatter(msg, dst, zinit):
    E, D = msg.shape
    NP = zinit.shape[0]
    NW = 32
    EPW = E // NW
    CH = 40
    KFULL = EPW // CH
    RPT = NP // 16

    mesh = plsc.VectorSubcoreMesh(core_axis_name="c", subcore_axis_name="s")

    @functools.partial(
        pl.kernel, mesh=mesh,
        compiler_params=pltpu.CompilerParams(use_tc_tiling_on_sc=False),
        out_type=jax.ShapeDtypeStruct((2, NP, D), jnp.float32),
        scratch_types=[
            pltpu.VMEM_SHARED((NP, D), jnp.float32),
            pltpu.VMEM((2, CH), jnp.int32),
            pltpu.VMEM((2, CH, D), jnp.float32),
            pltpu.SemaphoreType.DMA,
            pltpu.SemaphoreType.DMA,
        ],
    )
    def sk(msg_h, dst_h, zin_h, out_h,
           acc_s, idx_v, rows_v, s0, s1):
        cid = lax.axis_index("c")
        sid = lax.axis_index("s")
        wid = sid * 2 + cid
        base = pl.multiple_of(wid * EPW, 8)
        r0 = pl.multiple_of(sid * RPT, 8)

        pltpu.sync_copy(zin_h.at[pl.ds(r0, RPT)], acc_s.at[pl.ds(r0, RPT)])
        plsc.subcore_barrier()

        sems = [s0, s1]

        def body(i, carry):
            cps = []
            for b in range(2):
                k = i * 2 + b
                o = pl.multiple_of(base + k * CH, 8)
                cps.append(pltpu.async_copy(
                    dst_h.at[pl.ds(o, CH)], idx_v.at[b], sems[b]))
                cps.append(pltpu.async_copy(
                    msg_h.at[pl.ds(o, CH)], rows_v.at[b], sems[b]))
            for b in range(2):
                cps[2 * b].wait()
                cps[2 * b + 1].wait()
                pltpu.sync_copy(rows_v.at[b], acc_s.at[idx_v.at[b]], add=True)
            return carry

        lax.fori_loop(0, KFULL // 2, body, 0, unroll=False)

        plsc.subcore_barrier()
        pltpu.sync_copy(acc_s.at[pl.ds(r0, RPT)], out_h.at[cid, pl.ds(r0, RPT)])

    return sk(msg, dst, zinit)


import numpy as _np


def _block_weights(wh, wu, V, H):
    L = _np.zeros((3, 3 * V, V), _np.float32)
    R = _np.zeros((3, H, 96), _np.float32)
    pmat = _np.zeros((96, H), _np.float32)
    rmat = _np.zeros((V, 3 * V), _np.float32)
    for c in range(3):
        L[c, 3 * _np.arange(V) + c, _np.arange(V)] = 1.0
        R[c, _np.arange(H), 32 * c + _np.arange(H)] = 1.0
        pmat[32 * c + _np.arange(H), _np.arange(H)] = 1.0
        rmat[_np.arange(V), 3 * _np.arange(V) + c] = 1.0
    whvm = jnp.einsum('cav,vh,chb->ab', L, wh[:V], R)
    whunit = (jnp.einsum('h,chb->cb', wh[V], R) if wh.shape[0] > V else None)
    wuvm = jnp.einsum('chb,hu,cau->ba', R, wu, jnp.asarray(L))
    return (whvm, whunit, jnp.asarray(pmat), wuvm,
            jnp.asarray(rmat), jnp.asarray(rmat.T))


def kernel(scalar_feats, coord_feats, positions, edge_index, params):
    N, S = scalar_feats.shape
    V = coord_feats.shape[1]
    E = edge_index.shape[1]
    src = edge_index[0]
    dst = edge_index[1]

    coord_f = coord_feats.reshape(N, 3 * V)
    tbl = jnp.concatenate(
        [scalar_feats, coord_f, positions,
         jnp.zeros((N, 13), jnp.float32)], axis=1)
    posp = jnp.concatenate(
        [positions, jnp.zeros((N, 13), jnp.float32)], axis=1)

    g1, g2 = _sc_gather(tbl, posp, src, dst)

    pm = params['msg']
    w_out = pm['W_out']
    H1 = V + 1
    whvm_m, whunit_m, pmat_m, wuvm_m, rmat_m, _ = _block_weights(
        pm['Wh'], pm['Wu'], V, H1)
    TB = 8000
    msg = pl.pallas_call(
        _edge_kernel,
        grid=(E // TB,),
        in_specs=[
            pl.BlockSpec((TB, 192), lambda i: (i, 0)),
            pl.BlockSpec((TB, 16), lambda i: (i, 0)),
        ] + [pl.BlockSpec(s, lambda i: (0, 0)) for s in
             [(3, V), (S, S), (RBF_DIM, S), (H1, S), (3 * V, 96), (3, 96),
              (96, H1), (96, 3 * V), (V, 3 * V), (1, S), (S, V), (1, V)]],
        out_specs=pl.BlockSpec((TB, S + 3 * V), lambda i: (i, 0)),
        out_shape=jax.ShapeDtypeStruct((E, S + 3 * V), jnp.float32),
    )(g1, g2, jnp.asarray(_np.ones((3, V), _np.float32)),
      w_out[:S], w_out[S:S + RBF_DIM], w_out[S + RBF_DIM:],
      whvm_m, whunit_m, pmat_m, wuvm_m, rmat_m,
      pm['b_out'][None, :], pm['W_gate'], pm['b_gate'][None, :])

    NP = 16 * 632
    zinit = jnp.zeros((NP, S + 3 * V), jnp.float32)
    parts = _sc_scatter(msg, dst, zinit)

    pu = params['upd']
    wu_out = pu['W_out']
    whvm_u, _, pmat_u, wuvm_u, rmat_u, rtm_u = _block_weights(
        pu['Wh'], pu['Wu'], V, V)
    NB = 2000
    out_s, out_v = pl.pallas_call(
        _node_kernel,
        grid=(N // NB,),
        in_specs=[
            pl.BlockSpec((1, NB, S + 3 * V), lambda i: (0, i, 0)),
            pl.BlockSpec((1, NB, S + 3 * V), lambda i: (1, i, 0)),
            pl.BlockSpec((NB, S), lambda i: (i, 0)),
            pl.BlockSpec((NB, 3 * V), lambda i: (i, 0)),
        ] + [pl.BlockSpec(s, lambda i: (0, 0)) for s in
             [(S, S), (V, S), (3 * V, 96), (96, V), (96, 3 * V),
              (V, 3 * V), (3 * V, V), (1, S), (S, V), (1, V),
              (1, S), (1, S), (1, S), (1, S)]],
        out_specs=[
            pl.BlockSpec((NB, S), lambda i: (i, 0)),
            pl.BlockSpec((NB, 3 * V), lambda i: (i, 0)),
        ],
        out_shape=[
            jax.ShapeDtypeStruct((N, S), jnp.float32),
            jax.ShapeDtypeStruct((N, 3 * V), jnp.float32),
        ],
    )(parts, parts, scalar_feats, coord_f,
      wu_out[:S], wu_out[S:], whvm_u, pmat_u, wuvm_u, rmat_u, rtm_u,
      pu['b_out'][None, :], pu['W_gate'], pu['b_gate'][None, :],
      params['msg_ln']['gamma'][None, :], params['msg_ln']['beta'][None, :],
      params['upd_ln']['gamma'][None, :], params['upd_ln']['beta'][None, :])

    v2 = out_v.reshape(N, V, 3)
    return out_s, v2
